# baseline (speedup 1.0000x reference)
import functools

import jax
import jax.numpy as jnp
from jax import lax
from jax.experimental import pallas as pl
from jax.experimental.pallas import tpu as pltpu
from jax.experimental.pallas import tpu_sc as plsc

VOCAB = 100000
D = 64
B = 1024

TILE_V = 2048


try:
    _SC_INFO = plsc.get_sparse_core_info()
    _NC = _SC_INFO.num_cores
    _NS = _SC_INFO.num_subcores
except Exception:
    _NC, _NS = 2, 16
_NW = _NC * _NS
_B_PER_W = B // _NW


_NTILE = 4
_TILE_EDGE = (VOCAB // 128) * 128
_LAST_START = _TILE_EDGE // 128 - 1
_TAIL_ROWS = VOCAB - _TILE_EDGE


def _sc_gather_body(table_t_hbm, tail_hbm, idx_hbm, out_hbm,
                    xv, gbuf, ebuf, tailbuf, sems, tsem):
    wid = lax.axis_index("s") * _NC + lax.axis_index("c")
    base = wid * _B_PER_W
    pltpu.sync_copy(idx_hbm.at[pl.ds(base, _B_PER_W)], xv)
    pltpu.async_copy(tail_hbm, tailbuf, tsem).wait()

    lanes16 = lax.iota(jnp.int32, 16)
    handles = {}
    scalars = {}

    def splat(s):
        return jnp.broadcast_to(s, (16,))

    for j in range(_B_PER_W + _NTILE):
        if j >= _NTILE:
            k = j - _NTILE
            handles[k].wait()
            l_s, trow_s, sel_s = scalars[k]
            for c4 in range(D // 16):
                d_idx = lanes16 + 16 * c4
                mvals = plsc.load_gather(
                    gbuf, [splat(jnp.int32(k % _NTILE)), d_idx, splat(l_s)])
                tvals = plsc.load_gather(tailbuf, [splat(trow_s), d_idx])
                vals = jnp.where(splat(sel_s), tvals, mvals)
                ebuf[k, pl.ds(16 * c4, 16)] = vals
        if j < _B_PER_W:
            slot = j % _NTILE
            c, pos = divmod(j, 16)
            v = xv[pl.ds(16 * c, 16)]
            x_s = jnp.max(jnp.where(lanes16 == pos, v, -1))
            tc_s = jnp.minimum(lax.shift_right_logical(x_s, 7),
                               jnp.int32(_LAST_START))
            start = pl.multiple_of(tc_s * 128, 128)
            handles[j] = pltpu.async_copy(
                table_t_hbm.at[:, pl.ds(start, 128)],
                gbuf.at[slot], sems.at[slot])
            l_s = jnp.minimum(x_s - tc_s * 128, 127)
            trow_s = jnp.clip(x_s - _TILE_EDGE, 0, _TAIL_ROWS - 1)
            sel_s = x_s >= _TILE_EDGE
            scalars[j] = (l_s, trow_s, sel_s)

    pltpu.sync_copy(ebuf, out_hbm.at[pl.ds(base, _B_PER_W)])


@functools.lru_cache(maxsize=None)
def _sc_gather():
    return pl.kernel(
        _sc_gather_body,
        mesh=plsc.VectorSubcoreMesh(core_axis_name="c", subcore_axis_name="s"),
        out_type=jax.ShapeDtypeStruct((B, 128), jnp.float32),
        scratch_types=[
            pltpu.VMEM((_B_PER_W,), jnp.int32),
            pltpu.VMEM((_NTILE, D, 128), jnp.float32),
            pltpu.VMEM((_B_PER_W, 128), jnp.float32),
            pltpu.VMEM((_TAIL_ROWS, D), jnp.float32),
            pltpu.SemaphoreType.DMA((_NTILE,)),
            pltpu.SemaphoreType.DMA,
        ],
        compiler_params=pltpu.CompilerParams(needs_layout_passes=False),
    )


NBUF = 4
_NFULL = VOCAB // TILE_V
_TAIL = VOCAB - _NFULL * TILE_V
_NSTEP = _NFULL + (1 if _TAIL else 0)


def _matmul_body(e_ref, wt_ref, b_ref, out_hbm, buf, sems):
    i = pl.program_id(0)
    slot = lax.rem(i, NBUF)

    @pl.when(i >= NBUF)
    def _():
        pltpu.make_async_copy(
            buf.at[slot], out_hbm.at[pl.ds(0, TILE_V), :], sems.at[slot]
        ).wait()

    acc = lax.dot_general(
        wt_ref[...], e_ref[...],
        dimension_numbers=(((0,), (1,)), ((), ())),
        preferred_element_type=jnp.float32,
    )
    buf[slot] = acc + jnp.transpose(b_ref[...])

    @pl.when(i < _NFULL)
    def _():
        pltpu.make_async_copy(
            buf.at[slot], out_hbm.at[pl.ds(i * TILE_V, TILE_V), :],
            sems.at[slot],
        ).start()

    if _TAIL:
        @pl.when(i == _NFULL)
        def _():
            pltpu.make_async_copy(
                buf.at[slot, pl.ds(0, _TAIL), :],
                out_hbm.at[pl.ds(_NFULL * TILE_V, _TAIL), :],
                sems.at[slot],
            ).start()

    @pl.when(i == _NSTEP - 1)
    def _():
        for k in range(max(_NSTEP - NBUF, 0), _NSTEP):
            s = k % NBUF
            if _TAIL and k == _NFULL:
                pltpu.make_async_copy(
                    buf.at[s, pl.ds(0, _TAIL), :],
                    out_hbm.at[pl.ds(_NFULL * TILE_V, _TAIL), :],
                    sems.at[s],
                ).wait()
            else:
                pltpu.make_async_copy(
                    buf.at[s], out_hbm.at[pl.ds(0, TILE_V), :], sems.at[s]
                ).wait()


def _tc_project_t(e, W_T, b2):
    return pl.pallas_call(
        _matmul_body,
        grid=(_NSTEP,),
        in_specs=[
            pl.BlockSpec((B, D), lambda i: (0, 0)),
            pl.BlockSpec((D, TILE_V), lambda i: (0, i)),
            pl.BlockSpec((1, TILE_V), lambda i: (0, i)),
        ],
        out_specs=pl.BlockSpec(memory_space=pl.ANY),
        out_shape=jax.ShapeDtypeStruct((VOCAB, B), jnp.float32),
        scratch_shapes=[
            pltpu.VMEM((NBUF, TILE_V, B), jnp.float32),
            pltpu.SemaphoreType.DMA((NBUF,)),
        ],
        compiler_params=pltpu.CompilerParams(
            dimension_semantics=("arbitrary",),
        ),
    )(e, W_T, b2)


def kernel(x, emb_table, W, b):
    tail = lax.slice(emb_table, (_TILE_EDGE, 0), (VOCAB, D))
    e_pad = _sc_gather()(emb_table.T, tail, x.astype(jnp.int32))
    e = e_pad[:, :D]
    logits_t = _tc_project_t(e, W.T, b.reshape(1, VOCAB))
    return (logits_t.T, e)


# speedup vs baseline: 1.0069x; 1.0069x over previous
import functools

import jax
import jax.numpy as jnp
from jax import lax
from jax.experimental import pallas as pl
from jax.experimental.pallas import tpu as pltpu
from jax.experimental.pallas import tpu_sc as plsc

VOCAB = 100000
D = 64
B = 1024

TILE_V = 2048


try:
    _SC_INFO = plsc.get_sparse_core_info()
    _NC = _SC_INFO.num_cores
    _NS = _SC_INFO.num_subcores
except Exception:
    _NC, _NS = 2, 16
_NW = _NC * _NS
_B_PER_W = B // _NW


_NTILE = 6
_TILE_EDGE = (VOCAB // 128) * 128
_LAST_START = _TILE_EDGE // 128 - 1
_TAIL_ROWS = VOCAB - _TILE_EDGE


def _sc_gather_body(table_t_hbm, tail_hbm, idx_hbm, out_hbm,
                    xv, gbuf, ebuf, tailbuf, sems, tsem):
    wid = lax.axis_index("s") * _NC + lax.axis_index("c")
    base = wid * _B_PER_W
    pltpu.sync_copy(idx_hbm.at[pl.ds(base, _B_PER_W)], xv)
    pltpu.async_copy(tail_hbm, tailbuf, tsem).wait()

    lanes16 = lax.iota(jnp.int32, 16)
    handles = {}
    scalars = {}

    def splat(s):
        return jnp.broadcast_to(s, (16,))

    for j in range(_B_PER_W + _NTILE):
        if j >= _NTILE:
            k = j - _NTILE
            handles[k].wait()
            l_s, trow_s, sel_s = scalars[k]
            for c4 in range(D // 16):
                d_idx = lanes16 + 16 * c4
                mvals = plsc.load_gather(
                    gbuf, [splat(jnp.int32(k % _NTILE)), d_idx, splat(l_s)])
                tvals = plsc.load_gather(tailbuf, [splat(trow_s), d_idx])
                vals = jnp.where(splat(sel_s), tvals, mvals)
                ebuf[k, pl.ds(16 * c4, 16)] = vals
        if j < _B_PER_W:
            slot = j % _NTILE
            c, pos = divmod(j, 16)
            v = xv[pl.ds(16 * c, 16)]
            x_s = jnp.max(jnp.where(lanes16 == pos, v, -1))
            tc_s = jnp.minimum(lax.shift_right_logical(x_s, 7),
                               jnp.int32(_LAST_START))
            start = pl.multiple_of(tc_s * 128, 128)
            handles[j] = pltpu.async_copy(
                table_t_hbm.at[:, pl.ds(start, 128)],
                gbuf.at[slot], sems.at[slot])
            l_s = jnp.minimum(x_s - tc_s * 128, 127)
            trow_s = jnp.clip(x_s - _TILE_EDGE, 0, _TAIL_ROWS - 1)
            sel_s = x_s >= _TILE_EDGE
            scalars[j] = (l_s, trow_s, sel_s)

    pltpu.sync_copy(ebuf, out_hbm.at[pl.ds(base, _B_PER_W)])


@functools.lru_cache(maxsize=None)
def _sc_gather():
    return pl.kernel(
        _sc_gather_body,
        mesh=plsc.VectorSubcoreMesh(core_axis_name="c", subcore_axis_name="s"),
        out_type=jax.ShapeDtypeStruct((B, 128), jnp.float32),
        scratch_types=[
            pltpu.VMEM((_B_PER_W,), jnp.int32),
            pltpu.VMEM((_NTILE, D, 128), jnp.float32),
            pltpu.VMEM((_B_PER_W, 128), jnp.float32),
            pltpu.VMEM((_TAIL_ROWS, D), jnp.float32),
            pltpu.SemaphoreType.DMA((_NTILE,)),
            pltpu.SemaphoreType.DMA,
        ],
        compiler_params=pltpu.CompilerParams(needs_layout_passes=False),
    )


NBUF = 4
_NFULL = VOCAB // TILE_V
_TAIL = VOCAB - _NFULL * TILE_V
_NSTEP = _NFULL + (1 if _TAIL else 0)


def _matmul_body(e_ref, wt_ref, b_ref, out_hbm, buf, sems):
    i = pl.program_id(0)
    slot = lax.rem(i, NBUF)

    @pl.when(i >= NBUF)
    def _():
        pltpu.make_async_copy(
            buf.at[slot], out_hbm.at[pl.ds(0, TILE_V), :], sems.at[slot]
        ).wait()

    acc = lax.dot_general(
        wt_ref[...], e_ref[...],
        dimension_numbers=(((0,), (1,)), ((), ())),
        preferred_element_type=jnp.float32,
    )
    buf[slot] = acc + jnp.transpose(b_ref[...])

    @pl.when(i < _NFULL)
    def _():
        pltpu.make_async_copy(
            buf.at[slot], out_hbm.at[pl.ds(i * TILE_V, TILE_V), :],
            sems.at[slot],
        ).start()

    if _TAIL:
        @pl.when(i == _NFULL)
        def _():
            pltpu.make_async_copy(
                buf.at[slot, pl.ds(0, _TAIL), :],
                out_hbm.at[pl.ds(_NFULL * TILE_V, _TAIL), :],
                sems.at[slot],
            ).start()

    @pl.when(i == _NSTEP - 1)
    def _():
        for k in range(max(_NSTEP - NBUF, 0), _NSTEP):
            s = k % NBUF
            if _TAIL and k == _NFULL:
                pltpu.make_async_copy(
                    buf.at[s, pl.ds(0, _TAIL), :],
                    out_hbm.at[pl.ds(_NFULL * TILE_V, _TAIL), :],
                    sems.at[s],
                ).wait()
            else:
                pltpu.make_async_copy(
                    buf.at[s], out_hbm.at[pl.ds(0, TILE_V), :], sems.at[s]
                ).wait()


def _tc_project_t(e, W_T, b2):
    return pl.pallas_call(
        _matmul_body,
        grid=(_NSTEP,),
        in_specs=[
            pl.BlockSpec((B, D), lambda i: (0, 0)),
            pl.BlockSpec((D, TILE_V), lambda i: (0, i)),
            pl.BlockSpec((1, TILE_V), lambda i: (0, i)),
        ],
        out_specs=pl.BlockSpec(memory_space=pl.ANY),
        out_shape=jax.ShapeDtypeStruct((VOCAB, B), jnp.float32),
        scratch_shapes=[
            pltpu.VMEM((NBUF, TILE_V, B), jnp.float32),
            pltpu.SemaphoreType.DMA((NBUF,)),
        ],
        compiler_params=pltpu.CompilerParams(
            dimension_semantics=("arbitrary",),
        ),
    )(e, W_T, b2)


def kernel(x, emb_table, W, b):
    tail = lax.slice(emb_table, (_TILE_EDGE, 0), (VOCAB, D))
    e_pad = _sc_gather()(emb_table.T, tail, x.astype(jnp.int32))
    e = e_pad[:, :D]
    logits_t = _tc_project_t(e, W.T, b.reshape(1, VOCAB))
    return (logits_t.T, e)
